# baseline (speedup 1.0000x reference)
import functools

import jax
import jax.numpy as jnp
from jax import lax
from jax.experimental import pallas as pl
from jax.experimental.pallas import tpu as pltpu
from jax.experimental.pallas import tpu_sc as plsc

N, D, H, O = 10000, 128, 128, 128
E1, E2 = 320000, 100000

NC, NS = 2, 16
NW = NC * NS
GB1 = 40
NB1 = E1 // NW // GB1
E2P = 100352
GB2 = 32
NB2 = E2P // NW // GB2

F32 = jnp.float32
BF16 = jnp.bfloat16


def _tables_body(x, w0, w1a, w2a, w1b, w2b, wout, b0, b1a, b2a, b1b, b2b,
                 bout, out0, q1, r1, q2, m1, m2, b2s, zb1, zb2, c1, c2):
    dot = functools.partial(jnp.dot, preferred_element_type=F32)
    w0v = w0[...]
    woutv = wout[...]
    p = dot(x[...], w0v)
    o0 = dot(p, woutv) + (dot(b0[...], woutv) + bout[...])
    out0[...] = o0
    a1 = w1a[0:H, :]
    bb1 = w1a[H:2 * H, :]
    q1[...] = dot(p, a1)
    r1[...] = dot(o0, bb1)
    a2 = w2a[0:H, :]
    bb2 = w2a[H:2 * H, :]
    q2[...] = dot(p, a2)
    m1[...] = dot(w1b[...], woutv)
    m2[...] = dot(w2b[...], woutv)
    b2s[...] = bb2 * (1.0 / 3.0)
    zb1[...] = dot(b0[...], a1) + b1a[...]
    zb2[...] = dot(b0[...], a2) + b2a[...]
    c1[...] = dot(b1b[...], woutv) + bout[...]
    c2[...] = dot(b2b[...], woutv) + bout[...]


def _build_tables(x, w0, w1a, w2a, w1b, w2b, wout, b0, b1a, b2a, b1b, b2b,
                  bout):
    nblk = 5
    rows = N // nblk
    row_spec = pl.BlockSpec((rows, H), lambda i: (i, 0))
    fix = lambda shape: pl.BlockSpec(shape, lambda i: (0, 0))
    return pl.pallas_call(
        _tables_body,
        grid=(nblk,),
        in_specs=[
            row_spec,
            fix((H, H)),
            fix((2 * H, H)),
            fix((2 * H, H)),
            fix((H, H)),
            fix((H, H)),
            fix((H, O)),
            fix((1, H)), fix((1, H)), fix((1, H)), fix((1, H)), fix((1, H)),
            fix((1, O)),
        ],
        out_specs=[
            row_spec, row_spec, row_spec, row_spec,
            fix((H, O)), fix((H, O)), fix((H, H)),
            fix((1, H)), fix((1, H)), fix((1, O)), fix((1, O)),
        ],
        out_shape=[
            jax.ShapeDtypeStruct((N, O), F32),
            jax.ShapeDtypeStruct((N, H), F32),
            jax.ShapeDtypeStruct((N, H), F32),
            jax.ShapeDtypeStruct((N, H), F32),
            jax.ShapeDtypeStruct((H, O), F32),
            jax.ShapeDtypeStruct((H, O), F32),
            jax.ShapeDtypeStruct((H, H), F32),
            jax.ShapeDtypeStruct((1, H), F32),
            jax.ShapeDtypeStruct((1, H), F32),
            jax.ShapeDtypeStruct((1, O), F32),
            jax.ShapeDtypeStruct((1, O), F32),
        ],
    )(x, w0, w1a, w2a, w1b, w2b, wout, b0, b1a, b2a, b1b, b2b, bout)


def _pipelined_gather_sum(groups, sums, g0, gb, nblk,
                          idxb, rowb, outb, isem, gsem, wsem):
    def idx_prefetch(s, b):
        for g, (tab, idx_hbm, ar) in enumerate(groups):
            pltpu.async_copy(
                idx_hbm.at[pl.ds((g0 + b) * gb * ar, gb * ar)],
                idxb[s][g], isem[s])

    def wait_idx(s):
        for g, (tab, idx_hbm, ar) in enumerate(groups):
            pltpu.make_async_copy(idx_hbm.at[pl.ds(0, gb * ar)],
                                  idxb[s][g], isem[s]).wait()

    def fire(s, b):
        wait_idx(s)
        for g, (tab, idx_hbm, ar) in enumerate(groups):
            pltpu.async_copy(tab.at[idxb[s][g]], rowb[s][g], gsem[s])

    def wait_gathers(s):
        for g, (tab, idx_hbm, ar) in enumerate(groups):
            pltpu.make_async_copy(tab.at[idxb[s][g]], rowb[s][g],
                                  gsem[s]).wait()

    def wb(s, b):
        osl = pl.ds((g0 + b) * gb, gb)
        for oi, (gids, out) in enumerate(sums):
            pltpu.async_copy(outb[s][oi], out.at[osl], wsem[s])

    def wait_wb(s):
        for oi, (gids, out) in enumerate(sums):
            pltpu.make_async_copy(outb[s][oi], out.at[pl.ds(g0 * gb, gb)],
                                  wsem[s]).wait()

    def accumulate(s):
        for oi, (gids, out) in enumerate(sums):
            dst = outb[s][oi]

            def row(r2, carry):
                for p in range(2):
                    r = r2 * 2 + p
                    for c in range(H // 16):
                        cs = pl.ds(c * 16, 16)
                        acc = None
                        for g in gids:
                            ar = groups[g][2]
                            for k in range(ar):
                                v = rowb[s][g][r * ar + k, cs]
                                acc = v if acc is None else acc + v
                        dst[r, cs] = acc
                return carry

            lax.fori_loop(0, gb // 2, row, 0)

    idx_prefetch(0, 0)
    idx_prefetch(1, 1)
    fire(0, 0)

    def pair(k2, carry):
        for p in range(2):
            k = k2 * 2 + p
            s, s2 = p, 1 - p

            @pl.when(jnp.logical_and(k + 1 < nblk, k >= 1))
            def _():
                wait_wb(s2)

            @pl.when(k + 1 < nblk)
            def _():
                fire(s2, k + 1)

            @pl.when(k < nblk)
            def _():
                wait_gathers(s)

                @pl.when(k + 2 < nblk)
                def _():
                    idx_prefetch(s, k + 2)

                accumulate(s)
                wb(s, k)
        return carry

    lax.fori_loop(0, (nblk + 1) // 2, pair, 0)
    wait_wb(0)
    wait_wb(1)


def _gather1_body(q1, r1, idxc, idxb_h, out,
                  ic0, ib0, ic1, ib1, rc0, rb0, rc1, rb1, oz0, oz1,
                  is0, is1, gs0, gs1, ws0, ws1):
    wid = lax.axis_index("s") * NC + lax.axis_index("c")
    g0 = wid * NB1
    _pipelined_gather_sum(
        groups=[(q1, idxc, 2), (r1, idxb_h, 2)],
        sums=[([0, 1], out)],
        g0=g0, gb=GB1, nblk=NB1,
        idxb=[[ic0, ib0], [ic1, ib1]],
        rowb=[[rc0, rb0], [rc1, rb1]],
        outb=[[oz0], [oz1]],
        isem=[is0, is1], gsem=[gs0, gs1], wsem=[ws0, ws1])


def _gather2_body(q2, o1, idxc, idxb_h, qs, bs,
                  ic0, ib0, ic1, ib1, rc0, rb0, rc1, rb1,
                  oq0, ob0, oq1, ob1,
                  is0, is1, gs0, gs1, ws0, ws1):
    wid = lax.axis_index("s") * NC + lax.axis_index("c")
    g0 = wid * NB2
    _pipelined_gather_sum(
        groups=[(q2, idxc, 3), (o1, idxb_h, 3)],
        sums=[([0], qs), ([1], bs)],
        g0=g0, gb=GB2, nblk=NB2,
        idxb=[[ic0, ib0], [ic1, ib1]],
        rowb=[[rc0, rb0], [rc1, rb1]],
        outb=[[oq0, ob0], [oq1, ob1]],
        isem=[is0, is1], gsem=[gs0, gs1], wsem=[ws0, ws1])


_SC_MESH = plsc.VectorSubcoreMesh(core_axis_name="c", subcore_axis_name="s")


def _gather_sum1(q1, r1, idx1c, idx1b):
    return pl.kernel(
        _gather1_body,
        mesh=_SC_MESH,
        out_type=jax.ShapeDtypeStruct((E1, H), F32),
        scratch_types=[pltpu.VMEM((2 * GB1,), jnp.int32) for _ in range(4)]
        + [pltpu.VMEM((2 * GB1, H), F32) for _ in range(4)]
        + [pltpu.VMEM((GB1, H), F32) for _ in range(2)]
        + [pltpu.SemaphoreType.DMA for _ in range(6)],
    )(q1, r1, idx1c, idx1b)


def _gather_sum2(q2, o1, idx2c, idx2b):
    return pl.kernel(
        _gather2_body,
        mesh=_SC_MESH,
        out_type=(jax.ShapeDtypeStruct((E2P, H), F32),
                  jax.ShapeDtypeStruct((E2P, H), F32)),
        scratch_types=[pltpu.VMEM((3 * GB2,), jnp.int32) for _ in range(4)]
        + [pltpu.VMEM((3 * GB2, H), F32) for _ in range(4)]
        + [pltpu.VMEM((GB2, H), F32) for _ in range(4)]
        + [pltpu.SemaphoreType.DMA for _ in range(6)],
    )(q2, o1, idx2c, idx2b)


def _mlp1_body(zsum, zb, m1, c1, out):
    z = jnp.maximum(zsum[...].astype(F32) * 0.5 + zb[...], 0.0)
    out[...] = jnp.dot(z, m1[...], preferred_element_type=F32) + c1[...]


def _mlp1(zsum1, zb1, m1, c1):
    blk = 2000
    grid = E1 // blk
    return pl.pallas_call(
        _mlp1_body,
        grid=(grid,),
        in_specs=[
            pl.BlockSpec((blk, H), lambda i: (i, 0)),
            pl.BlockSpec((1, H), lambda i: (0, 0)),
            pl.BlockSpec((H, O), lambda i: (0, 0)),
            pl.BlockSpec((1, O), lambda i: (0, 0)),
        ],
        out_specs=pl.BlockSpec((blk, O), lambda i: (i, 0)),
        out_shape=jax.ShapeDtypeStruct((E1, O), F32),
    )(zsum1, zb1, m1, c1)


def _mlp2_body(qs, bs, b2s, zb2, m2, c2, out):
    z = qs[...].astype(F32) * (1.0 / 3.0) + jnp.dot(
        bs[...], b2s[...], preferred_element_type=F32) + zb2[...]
    z = jnp.maximum(z, 0.0)
    out[...] = jnp.dot(z, m2[...], preferred_element_type=F32) + c2[...]


def _mlp2(qsum2, bsum2, b2s, zb2, m2, c2):
    blk = 2048
    grid = E2P // blk
    return pl.pallas_call(
        _mlp2_body,
        grid=(grid,),
        in_specs=[
            pl.BlockSpec((blk, H), lambda i: (i, 0)),
            pl.BlockSpec((blk, H), lambda i: (i, 0)),
            pl.BlockSpec((H, H), lambda i: (0, 0)),
            pl.BlockSpec((1, H), lambda i: (0, 0)),
            pl.BlockSpec((H, O), lambda i: (0, 0)),
            pl.BlockSpec((1, O), lambda i: (0, 0)),
        ],
        out_specs=pl.BlockSpec((blk, O), lambda i: (i, 0)),
        out_shape=jax.ShapeDtypeStruct((E2P, O), F32),
    )(qsum2, bsum2, b2s, zb2, m2, c2)


def kernel(chunk_features, W0, b0, W1a, b1a, W1b, b1b, W2a, b2a, W2b, b2b,
           Wout, bout, cell1_chunk_idx, cell1_boundary_idx, cell2_chunk_idx,
           cell2_boundary_idx):
    row = lambda v: v.reshape(1, -1)
    (out0, q1, r1, q2, m1, m2, b2s, zb1, zb2, c1, c2) = _build_tables(
        chunk_features, W0, W1a, W2a, W1b, W2b, Wout,
        row(b0), row(b1a), row(b2a), row(b1b), row(b2b), row(bout))

    i32 = jnp.int32
    zsum1 = _gather_sum1(q1, r1,
                         cell1_chunk_idx.astype(i32).reshape(-1),
                         cell1_boundary_idx.astype(i32).reshape(-1))
    out1 = _mlp1(zsum1, zb1, m1, c1)

    pad = ((0, E2P - E2), (0, 0))
    qsum2, bsum2 = _gather_sum2(
        q2, out1,
        jnp.pad(cell2_chunk_idx.astype(i32), pad).reshape(-1),
        jnp.pad(cell2_boundary_idx.astype(i32), pad).reshape(-1))
    out2 = _mlp2(qsum2, bsum2, b2s, zb2, m2, c2)[:E2]

    return (out0, out1, out2)


# speedup vs baseline: 2.1183x; 2.1183x over previous
import functools

import jax
import jax.numpy as jnp
from jax import lax
from jax.experimental import pallas as pl
from jax.experimental.pallas import tpu as pltpu
from jax.experimental.pallas import tpu_sc as plsc

N, D, H, O = 10000, 128, 128, 128
E1, E2 = 320000, 100000

NC, NS = 2, 16
NW = NC * NS
GB1 = 80
NB1 = E1 // NW // GB1
E2P = 100352
GB2 = 56
NB2 = E2P // NW // GB2

F32 = jnp.float32
BF16 = jnp.bfloat16


def _tables_body(x, w0, w1a, w2a, w1b, w2b, wout, b0, b1a, b2a, b1b, b2b,
                 bout, out0, q1, r1, q2, m1, m2, b2s, zb1, zb2, c1, c2):
    dot = functools.partial(jnp.dot, preferred_element_type=F32)
    w0v = w0[...]
    woutv = wout[...]
    p = dot(x[...], w0v)
    o0 = dot(p, woutv) + (dot(b0[...], woutv) + bout[...])
    out0[...] = o0
    a1 = w1a[0:H, :]
    bb1 = w1a[H:2 * H, :]
    q1[...] = dot(p, a1)
    r1[...] = dot(o0, bb1)
    a2 = w2a[0:H, :]
    bb2 = w2a[H:2 * H, :]
    q2[...] = dot(p, a2)
    m1[...] = dot(w1b[...], woutv)
    m2[...] = dot(w2b[...], woutv)
    b2s[...] = bb2 * (1.0 / 3.0)
    zb1[...] = dot(b0[...], a1) + b1a[...]
    zb2[...] = dot(b0[...], a2) + b2a[...]
    c1[...] = dot(b1b[...], woutv) + bout[...]
    c2[...] = dot(b2b[...], woutv) + bout[...]


def _build_tables(x, w0, w1a, w2a, w1b, w2b, wout, b0, b1a, b2a, b1b, b2b,
                  bout):
    nblk = 5
    rows = N // nblk
    row_spec = pl.BlockSpec((rows, H), lambda i: (i, 0))
    fix = lambda shape: pl.BlockSpec(shape, lambda i: (0, 0))
    return pl.pallas_call(
        _tables_body,
        grid=(nblk,),
        in_specs=[
            row_spec,
            fix((H, H)),
            fix((2 * H, H)),
            fix((2 * H, H)),
            fix((H, H)),
            fix((H, H)),
            fix((H, O)),
            fix((1, H)), fix((1, H)), fix((1, H)), fix((1, H)), fix((1, H)),
            fix((1, O)),
        ],
        out_specs=[
            row_spec, row_spec, row_spec, row_spec,
            fix((H, O)), fix((H, O)), fix((H, H)),
            fix((1, H)), fix((1, H)), fix((1, O)), fix((1, O)),
        ],
        out_shape=[
            jax.ShapeDtypeStruct((N, O), F32),
            jax.ShapeDtypeStruct((N, H), F32),
            jax.ShapeDtypeStruct((N, H), F32),
            jax.ShapeDtypeStruct((N, H), F32),
            jax.ShapeDtypeStruct((H, O), F32),
            jax.ShapeDtypeStruct((H, O), F32),
            jax.ShapeDtypeStruct((H, H), F32),
            jax.ShapeDtypeStruct((1, H), F32),
            jax.ShapeDtypeStruct((1, H), F32),
            jax.ShapeDtypeStruct((1, O), F32),
            jax.ShapeDtypeStruct((1, O), F32),
        ],
    )(x, w0, w1a, w2a, w1b, w2b, wout, b0, b1a, b2a, b1b, b2b, bout)


def _acc_rows(bufs, nrows):
    nb = len(bufs)

    def row(r2, carry):
        for p in range(2):
            r = r2 * 2 + p
            for c in range(H // 16):
                s = pl.ds(c * 16, 16)
                acc = bufs[0][r, s]
                for k in range(1, nb):
                    acc = acc + bufs[k][r, s]
                bufs[0][r, s] = acc
        return carry

    lax.fori_loop(0, nrows // 2, row, 0)


def _pipelined_gather_sum(tables, sums, idx_hbm, g0, gb, nblk,
                          idxb, rowb, isem, gsem, wsem):
    nslots = len(tables)
    e_total = idx_hbm.shape[0] // nslots

    def idx_prefetch(s, b):
        for t in range(nslots):
            pltpu.async_copy(
                idx_hbm.at[pl.ds(t * e_total + (g0 + b) * gb, gb)],
                idxb[s].at[t], isem[s])

    def wait_idx(s):
        for t in range(nslots):
            pltpu.make_async_copy(idx_hbm.at[pl.ds(0, gb)],
                                  idxb[s].at[t], isem[s]).wait()

    def fire(s, b):
        wait_idx(s)
        for t, tab in enumerate(tables):
            pltpu.async_copy(tab.at[idxb[s].at[t]], rowb[s][t], gsem[s])

    def wait_gathers(s):
        for t, tab in enumerate(tables):
            pltpu.make_async_copy(tab.at[idxb[s].at[t]], rowb[s][t],
                                  gsem[s]).wait()

    def wb(s, b):
        osl = pl.ds((g0 + b) * gb, gb)
        for slots, out in sums:
            pltpu.async_copy(rowb[s][slots[0]], out.at[osl], wsem[s])

    def wait_wb(s):
        for slots, out in sums:
            pltpu.make_async_copy(rowb[s][slots[0]],
                                  out.at[pl.ds(0, gb)], wsem[s]).wait()

    idx_prefetch(0, 0)
    idx_prefetch(1, 1)
    fire(0, 0)

    def pair(k2, carry):
        for p in range(2):
            k = k2 * 2 + p
            s, s2 = p, 1 - p

            @pl.when(jnp.logical_and(k + 1 < nblk, k >= 1))
            def _():
                wait_wb(s2)

            @pl.when(k + 1 < nblk)
            def _():
                fire(s2, k + 1)

            @pl.when(k < nblk)
            def _():
                wait_gathers(s)

                @pl.when(k + 2 < nblk)
                def _():
                    idx_prefetch(s, k + 2)

                for slots, out in sums:
                    _acc_rows([rowb[s][sl] for sl in slots], gb)
                wb(s, k)
        return carry

    lax.fori_loop(0, (nblk + 1) // 2, pair, 0)
    wait_wb(0)
    wait_wb(1)


def _gather1_body(q1, r1, idx, out,
                  ix0, ix1, ra0, rb0, rc0, rd0, ra1, rb1, rc1, rd1,
                  is0, is1, gs0, gs1, ws0, ws1):
    wid = lax.axis_index("s") * NC + lax.axis_index("c")
    g0 = wid * NB1
    _pipelined_gather_sum(
        tables=[q1, q1, r1, r1],
        sums=[([0, 1, 2, 3], out)],
        idx_hbm=idx,
        g0=g0, gb=GB1, nblk=NB1,
        idxb=[ix0, ix1],
        rowb=[[ra0, rb0, rc0, rd0], [ra1, rb1, rc1, rd1]],
        isem=[is0, is1], gsem=[gs0, gs1], wsem=[ws0, ws1])


def _gather2_body(q2, o1, idx, qs, bs,
                  ix0, ix1, ga0, gb0, gc0, ha0, hb0, hc0,
                  ga1, gb1, gc1, ha1, hb1, hc1,
                  is0, is1, gs0, gs1, ws0, ws1):
    wid = lax.axis_index("s") * NC + lax.axis_index("c")
    g0 = wid * NB2
    _pipelined_gather_sum(
        tables=[q2, q2, q2, o1, o1, o1],
        sums=[([0, 1, 2], qs), ([3, 4, 5], bs)],
        idx_hbm=idx,
        g0=g0, gb=GB2, nblk=NB2,
        idxb=[ix0, ix1],
        rowb=[[ga0, gb0, gc0, ha0, hb0, hc0],
              [ga1, gb1, gc1, ha1, hb1, hc1]],
        isem=[is0, is1], gsem=[gs0, gs1], wsem=[ws0, ws1])


_SC_MESH = plsc.VectorSubcoreMesh(core_axis_name="c", subcore_axis_name="s")


def _gather_sum1(q1, r1, idx1):
    return pl.kernel(
        _gather1_body,
        mesh=_SC_MESH,
        out_type=jax.ShapeDtypeStruct((E1, H), F32),
        scratch_types=[pltpu.VMEM((4, GB1), jnp.int32) for _ in range(2)]
        + [pltpu.VMEM((GB1, H), F32) for _ in range(8)]
        + [pltpu.SemaphoreType.DMA for _ in range(6)],
    )(q1, r1, idx1)


def _gather_sum2(q2, o1, idx2):
    return pl.kernel(
        _gather2_body,
        mesh=_SC_MESH,
        out_type=(jax.ShapeDtypeStruct((E2P, H), F32),
                  jax.ShapeDtypeStruct((E2P, H), F32)),
        scratch_types=[pltpu.VMEM((6, GB2), jnp.int32) for _ in range(2)]
        + [pltpu.VMEM((GB2, H), F32) for _ in range(12)]
        + [pltpu.SemaphoreType.DMA for _ in range(6)],
    )(q2, o1, idx2)


def _mlp1_body(zsum, zb, m1, c1, out):
    z = jnp.maximum(zsum[...].astype(F32) * 0.5 + zb[...], 0.0)
    out[...] = jnp.dot(z, m1[...], preferred_element_type=F32) + c1[...]


def _mlp1(zsum1, zb1, m1, c1):
    blk = 2000
    grid = E1 // blk
    return pl.pallas_call(
        _mlp1_body,
        grid=(grid,),
        in_specs=[
            pl.BlockSpec((blk, H), lambda i: (i, 0)),
            pl.BlockSpec((1, H), lambda i: (0, 0)),
            pl.BlockSpec((H, O), lambda i: (0, 0)),
            pl.BlockSpec((1, O), lambda i: (0, 0)),
        ],
        out_specs=pl.BlockSpec((blk, O), lambda i: (i, 0)),
        out_shape=jax.ShapeDtypeStruct((E1, O), F32),
    )(zsum1, zb1, m1, c1)


def _mlp2_body(qs, bs, b2s, zb2, m2, c2, out):
    z = qs[...].astype(F32) * (1.0 / 3.0) + jnp.dot(
        bs[...], b2s[...], preferred_element_type=F32) + zb2[...]
    z = jnp.maximum(z, 0.0)
    out[...] = jnp.dot(z, m2[...], preferred_element_type=F32) + c2[...]


def _mlp2(qsum2, bsum2, b2s, zb2, m2, c2):
    blk = 2048
    grid = E2P // blk
    return pl.pallas_call(
        _mlp2_body,
        grid=(grid,),
        in_specs=[
            pl.BlockSpec((blk, H), lambda i: (i, 0)),
            pl.BlockSpec((blk, H), lambda i: (i, 0)),
            pl.BlockSpec((H, H), lambda i: (0, 0)),
            pl.BlockSpec((1, H), lambda i: (0, 0)),
            pl.BlockSpec((H, O), lambda i: (0, 0)),
            pl.BlockSpec((1, O), lambda i: (0, 0)),
        ],
        out_specs=pl.BlockSpec((blk, O), lambda i: (i, 0)),
        out_shape=jax.ShapeDtypeStruct((E2P, O), F32),
    )(qsum2, bsum2, b2s, zb2, m2, c2)


def kernel(chunk_features, W0, b0, W1a, b1a, W1b, b1b, W2a, b2a, W2b, b2b,
           Wout, bout, cell1_chunk_idx, cell1_boundary_idx, cell2_chunk_idx,
           cell2_boundary_idx):
    row = lambda v: v.reshape(1, -1)
    (out0, q1, r1, q2, m1, m2, b2s, zb1, zb2, c1, c2) = _build_tables(
        chunk_features, W0, W1a, W2a, W1b, W2b, Wout,
        row(b0), row(b1a), row(b2a), row(b1b), row(b2b), row(bout))

    i32 = jnp.int32
    idx1 = jnp.concatenate(
        [cell1_chunk_idx.astype(i32).T, cell1_boundary_idx.astype(i32).T],
        axis=0).reshape(-1)
    zsum1 = _gather_sum1(q1, r1, idx1)
    out1 = _mlp1(zsum1, zb1, m1, c1)

    pad = E2P - E2
    idx2 = jnp.pad(
        jnp.concatenate([cell2_chunk_idx.astype(i32).T,
                         cell2_boundary_idx.astype(i32).T], axis=0),
        ((0, 0), (0, pad))).reshape(-1)
    qsum2, bsum2 = _gather_sum2(q2, out1, idx2)
    out2 = _mlp2(qsum2, bsum2, b2s, zb2, m2, c2)[:E2]

    return (out0, out1, out2)


# speedup vs baseline: 2.1321x; 1.0065x over previous
import functools

import jax
import jax.numpy as jnp
from jax import lax
from jax.experimental import pallas as pl
from jax.experimental.pallas import tpu as pltpu
from jax.experimental.pallas import tpu_sc as plsc

N, D, H, O = 10000, 128, 128, 128
E1, E2 = 320000, 100000

NC, NS = 2, 16
NW = NC * NS
GB1 = 80
NB1 = E1 // NW // GB1
E2P = 100352
GB2 = 112
NB2 = E2P // NW // GB2

F32 = jnp.float32
BF16 = jnp.bfloat16


def _tables_body(x, w0, w1a, w2a, w1b, w2b, wout, b0, b1a, b2a, b1b, b2b,
                 bout, out0, q1, r1, q2, m1, m2, b2s, zb1, zb2, c1, c2):
    dot = functools.partial(jnp.dot, preferred_element_type=F32)
    w0v = w0[...]
    woutv = wout[...]
    p = dot(x[...], w0v)
    o0 = dot(p, woutv) + (dot(b0[...], woutv) + bout[...])
    out0[...] = o0
    a1 = w1a[0:H, :]
    bb1 = w1a[H:2 * H, :]
    q1[...] = dot(p, a1)
    r1[...] = dot(o0, bb1)
    a2 = w2a[0:H, :]
    bb2 = w2a[H:2 * H, :]
    q2[...] = dot(p, a2)
    m1[...] = dot(w1b[...], woutv)
    m2[...] = dot(w2b[...], woutv)
    b2s[...] = bb2 * (1.0 / 3.0)
    zb1[...] = dot(b0[...], a1) + b1a[...]
    zb2[...] = dot(b0[...], a2) + b2a[...]
    c1[...] = dot(b1b[...], woutv) + bout[...]
    c2[...] = dot(b2b[...], woutv) + bout[...]


def _build_tables(x, w0, w1a, w2a, w1b, w2b, wout, b0, b1a, b2a, b1b, b2b,
                  bout):
    nblk = 5
    rows = N // nblk
    row_spec = pl.BlockSpec((rows, H), lambda i: (i, 0))
    fix = lambda shape: pl.BlockSpec(shape, lambda i: (0, 0))
    return pl.pallas_call(
        _tables_body,
        grid=(nblk,),
        in_specs=[
            row_spec,
            fix((H, H)),
            fix((2 * H, H)),
            fix((2 * H, H)),
            fix((H, H)),
            fix((H, H)),
            fix((H, O)),
            fix((1, H)), fix((1, H)), fix((1, H)), fix((1, H)), fix((1, H)),
            fix((1, O)),
        ],
        out_specs=[
            row_spec, row_spec, row_spec, row_spec,
            fix((H, O)), fix((H, O)), fix((H, H)),
            fix((1, H)), fix((1, H)), fix((1, O)), fix((1, O)),
        ],
        out_shape=[
            jax.ShapeDtypeStruct((N, O), F32),
            jax.ShapeDtypeStruct((N, H), F32),
            jax.ShapeDtypeStruct((N, H), F32),
            jax.ShapeDtypeStruct((N, H), F32),
            jax.ShapeDtypeStruct((H, O), F32),
            jax.ShapeDtypeStruct((H, O), F32),
            jax.ShapeDtypeStruct((H, H), F32),
            jax.ShapeDtypeStruct((1, H), F32),
            jax.ShapeDtypeStruct((1, H), F32),
            jax.ShapeDtypeStruct((1, O), F32),
            jax.ShapeDtypeStruct((1, O), F32),
        ],
    )(x, w0, w1a, w2a, w1b, w2b, wout, b0, b1a, b2a, b1b, b2b, bout)


def _acc_rows(bufs, nrows):
    nb = len(bufs)

    def row(r2, carry):
        for p in range(2):
            r = r2 * 2 + p
            for c in range(H // 16):
                s = pl.ds(c * 16, 16)
                acc = bufs[0][r, s]
                for k in range(1, nb):
                    acc = acc + bufs[k][r, s]
                bufs[0][r, s] = acc
        return carry

    lax.fori_loop(0, nrows // 2, row, 0)


def _pipelined_gather_sum(tables, sums, idx_hbm, g0, gb, nblk,
                          idxb, rowb, isem, gsem, wsem):
    nslots = len(tables)
    e_total = idx_hbm.shape[0] // nslots

    def idx_prefetch(s, b):
        for t in range(nslots):
            pltpu.async_copy(
                idx_hbm.at[pl.ds(t * e_total + (g0 + b) * gb, gb)],
                idxb[s].at[t], isem[s])

    def wait_idx(s):
        for t in range(nslots):
            pltpu.make_async_copy(idx_hbm.at[pl.ds(0, gb)],
                                  idxb[s].at[t], isem[s]).wait()

    def fire(s, b):
        wait_idx(s)
        for t, tab in enumerate(tables):
            pltpu.async_copy(tab.at[idxb[s].at[t]], rowb[s][t], gsem[s])

    def wait_gathers(s):
        for t, tab in enumerate(tables):
            pltpu.make_async_copy(tab.at[idxb[s].at[t]], rowb[s][t],
                                  gsem[s]).wait()

    def wb(s, b):
        osl = pl.ds((g0 + b) * gb, gb)
        for slots, out in sums:
            pltpu.async_copy(rowb[s][slots[0]], out.at[osl], wsem[s])

    def wait_wb(s):
        for slots, out in sums:
            pltpu.make_async_copy(rowb[s][slots[0]],
                                  out.at[pl.ds(0, gb)], wsem[s]).wait()

    idx_prefetch(0, 0)
    idx_prefetch(1, 1)
    fire(0, 0)

    def pair(k2, carry):
        for p in range(2):
            k = k2 * 2 + p
            s, s2 = p, 1 - p

            @pl.when(jnp.logical_and(k + 1 < nblk, k >= 1))
            def _():
                wait_wb(s2)

            @pl.when(k + 1 < nblk)
            def _():
                fire(s2, k + 1)

            @pl.when(k < nblk)
            def _():
                wait_gathers(s)

                @pl.when(k + 2 < nblk)
                def _():
                    idx_prefetch(s, k + 2)

                for slots, out in sums:
                    _acc_rows([rowb[s][sl] for sl in slots], gb)
                wb(s, k)
        return carry

    lax.fori_loop(0, (nblk + 1) // 2, pair, 0)
    wait_wb(0)
    wait_wb(1)


def _gather1_body(q1, r1, idx, out,
                  ix0, ix1, ra0, rb0, rc0, rd0, ra1, rb1, rc1, rd1,
                  is0, is1, gs0, gs1, ws0, ws1):
    wid = lax.axis_index("s") * NC + lax.axis_index("c")
    g0 = wid * NB1
    _pipelined_gather_sum(
        tables=[q1, q1, r1, r1],
        sums=[([0, 1, 2, 3], out)],
        idx_hbm=idx,
        g0=g0, gb=GB1, nblk=NB1,
        idxb=[ix0, ix1],
        rowb=[[ra0, rb0, rc0, rd0], [ra1, rb1, rc1, rd1]],
        isem=[is0, is1], gsem=[gs0, gs1], wsem=[ws0, ws1])


def _gather2_body(tab, idx, out,
                  ix0, ix1, ga0, gb0, gc0, ga1, gb1, gc1,
                  is0, is1, gs0, gs1, ws0, ws1):
    wid = lax.axis_index("s") * NC + lax.axis_index("c")
    g0 = wid * NB2
    _pipelined_gather_sum(
        tables=[tab, tab, tab],
        sums=[([0, 1, 2], out)],
        idx_hbm=idx,
        g0=g0, gb=GB2, nblk=NB2,
        idxb=[ix0, ix1],
        rowb=[[ga0, gb0, gc0], [ga1, gb1, gc1]],
        isem=[is0, is1], gsem=[gs0, gs1], wsem=[ws0, ws1])


_SC_MESH = plsc.VectorSubcoreMesh(core_axis_name="c", subcore_axis_name="s")


def _gather_sum1(q1, r1, idx1):
    return pl.kernel(
        _gather1_body,
        mesh=_SC_MESH,
        out_type=jax.ShapeDtypeStruct((E1, H), F32),
        scratch_types=[pltpu.VMEM((4, GB1), jnp.int32) for _ in range(2)]
        + [pltpu.VMEM((GB1, H), F32) for _ in range(8)]
        + [pltpu.SemaphoreType.DMA for _ in range(6)],
    )(q1, r1, idx1)


def _gather_sum2(tab, idx2half):
    return pl.kernel(
        _gather2_body,
        mesh=_SC_MESH,
        out_type=jax.ShapeDtypeStruct((E2P, H), F32),
        scratch_types=[pltpu.VMEM((3, GB2), jnp.int32) for _ in range(2)]
        + [pltpu.VMEM((GB2, H), F32) for _ in range(6)]
        + [pltpu.SemaphoreType.DMA for _ in range(6)],
    )(tab, idx2half)


def _mlp1_body(zsum, zb, m1, c1, out):
    z = jnp.maximum(zsum[...].astype(F32) * 0.5 + zb[...], 0.0)
    out[...] = jnp.dot(z, m1[...], preferred_element_type=F32) + c1[...]


def _mlp1(zsum1, zb1, m1, c1):
    blk = 2000
    grid = E1 // blk
    return pl.pallas_call(
        _mlp1_body,
        grid=(grid,),
        in_specs=[
            pl.BlockSpec((blk, H), lambda i: (i, 0)),
            pl.BlockSpec((1, H), lambda i: (0, 0)),
            pl.BlockSpec((H, O), lambda i: (0, 0)),
            pl.BlockSpec((1, O), lambda i: (0, 0)),
        ],
        out_specs=pl.BlockSpec((blk, O), lambda i: (i, 0)),
        out_shape=jax.ShapeDtypeStruct((E1, O), F32),
    )(zsum1, zb1, m1, c1)


def _mlp2_body(qs, bs, b2s, zb2, m2, c2, out):
    z = qs[...].astype(F32) * (1.0 / 3.0) + jnp.dot(
        bs[...], b2s[...], preferred_element_type=F32) + zb2[...]
    z = jnp.maximum(z, 0.0)
    out[...] = jnp.dot(z, m2[...], preferred_element_type=F32) + c2[...]


def _mlp2(qsum2, bsum2, b2s, zb2, m2, c2):
    blk = 2048
    grid = E2P // blk
    return pl.pallas_call(
        _mlp2_body,
        grid=(grid,),
        in_specs=[
            pl.BlockSpec((blk, H), lambda i: (i, 0)),
            pl.BlockSpec((blk, H), lambda i: (i, 0)),
            pl.BlockSpec((H, H), lambda i: (0, 0)),
            pl.BlockSpec((1, H), lambda i: (0, 0)),
            pl.BlockSpec((H, O), lambda i: (0, 0)),
            pl.BlockSpec((1, O), lambda i: (0, 0)),
        ],
        out_specs=pl.BlockSpec((blk, O), lambda i: (i, 0)),
        out_shape=jax.ShapeDtypeStruct((E2P, O), F32),
    )(qsum2, bsum2, b2s, zb2, m2, c2)


def kernel(chunk_features, W0, b0, W1a, b1a, W1b, b1b, W2a, b2a, W2b, b2b,
           Wout, bout, cell1_chunk_idx, cell1_boundary_idx, cell2_chunk_idx,
           cell2_boundary_idx):
    row = lambda v: v.reshape(1, -1)
    (out0, q1, r1, q2, m1, m2, b2s, zb1, zb2, c1, c2) = _build_tables(
        chunk_features, W0, W1a, W2a, W1b, W2b, Wout,
        row(b0), row(b1a), row(b2a), row(b1b), row(b2b), row(bout))

    i32 = jnp.int32
    idx1 = jnp.concatenate(
        [cell1_chunk_idx.astype(i32).T, cell1_boundary_idx.astype(i32).T],
        axis=0).reshape(-1)
    zsum1 = _gather_sum1(q1, r1, idx1)
    out1 = _mlp1(zsum1, zb1, m1, c1)

    pad = E2P - E2
    pidx = lambda a: jnp.pad(a.astype(i32).T,
                             ((0, 0), (0, pad))).reshape(-1)
    qsum2 = _gather_sum2(q2, pidx(cell2_chunk_idx))
    bsum2 = _gather_sum2(out1, pidx(cell2_boundary_idx))
    out2 = _mlp2(qsum2, bsum2, b2s, zb2, m2, c2)[:E2]

    return (out0, out1, out2)
